# Initial kernel scaffold; baseline (speedup 1.0000x reference)
import jax
import jax.numpy as jnp
from jax.experimental import pallas as pl


def kernel(node, edge, adj, g1, b1, g2, b2, w11, bb11, w12, bb12, w21, bb21, w22, bb22):
    raise NotImplementedError("write your pallas kernel here")


# speedup vs baseline: 3.5266x; 3.5266x over previous
import functools

import jax
import jax.numpy as jnp
from jax.experimental import pallas as pl
from jax.experimental.pallas import tpu as pltpu


def _gelu(x):
    return 0.5 * x * (1.0 + jax.lax.erf(x * 0.7071067811865476))


def _stage1_kernel(node_ref, g1_ref, b1_ref, w11_ref, bb11_ref, w12_ref,
                   bb12_ref, x1_ref, x1h_ref):
    x = node_ref[...]
    m = jnp.mean(x, axis=-1, keepdims=True)
    v = jnp.mean((x - m) ** 2, axis=-1, keepdims=True)
    xn = (x - m) * jax.lax.rsqrt(v + 1e-5) * g1_ref[...] + b1_ref[...]
    h = _gelu(
        jnp.dot(xn, w11_ref[...], preferred_element_type=jnp.float32)
        + bb11_ref[...])
    x1 = (jnp.dot(h, w12_ref[...], preferred_element_type=jnp.float32)
          + bb12_ref[...] + x)
    x1_ref[...] = x1
    x1h_ref[...] = x1.astype(jnp.bfloat16)


def _stage2_kernel(adj_ref, x1h_ref, x1_ref, g2_ref, b2_ref, w21_ref,
                   bb21_ref, w22_ref, bb22_ref, out_ref):
    C = x1_ref.shape[-1]
    adj16 = adj_ref[...].astype(jnp.bfloat16)
    agg = jnp.dot(adj16, x1h_ref[0], preferred_element_type=jnp.float32)
    x1 = x1_ref[0]
    s = jnp.sum(x1, axis=-1, keepdims=True) + jnp.sum(agg, axis=-1,
                                                      keepdims=True)
    m = s / (2 * C)
    q = (jnp.sum((x1 - m) ** 2, axis=-1, keepdims=True)
         + jnp.sum((agg - m) ** 2, axis=-1, keepdims=True))
    rs = jax.lax.rsqrt(q / (2 * C) + 1e-5)
    g2 = g2_ref[...]
    b2 = b2_ref[...]
    xa = (x1 - m) * rs * g2[:, :C] + b2[:, :C]
    xb = (agg - m) * rs * g2[:, C:] + b2[:, C:]
    w21 = w21_ref[...]
    h = _gelu(
        jnp.dot(xa, w21[:C], preferred_element_type=jnp.float32)
        + jnp.dot(xb, w21[C:], preferred_element_type=jnp.float32)
        + bb21_ref[...])
    out_ref[0] = (jnp.dot(h, w22_ref[...], preferred_element_type=jnp.float32)
                  + bb22_ref[...] + x1)


@functools.partial(jax.jit, static_argnames=())
def kernel(node, edge, adj, g1, b1, g2, b2, w11, bb11, w12, bb12, w21, bb21,
           w22, bb22):
    B, N, C = node.shape
    H = w11.shape[1]
    flat = node.reshape(B * N, C)

    TN = 1024
    rep = lambda *_: (0, 0)
    x1_flat, x1h_flat = pl.pallas_call(
        _stage1_kernel,
        grid=(B * N // TN,),
        in_specs=[
            pl.BlockSpec((TN, C), lambda i: (i, 0)),
            pl.BlockSpec((1, C), rep),
            pl.BlockSpec((1, C), rep),
            pl.BlockSpec((C, H), rep),
            pl.BlockSpec((1, H), rep),
            pl.BlockSpec((H, C), rep),
            pl.BlockSpec((1, C), rep),
        ],
        out_specs=[
            pl.BlockSpec((TN, C), lambda i: (i, 0)),
            pl.BlockSpec((TN, C), lambda i: (i, 0)),
        ],
        out_shape=[
            jax.ShapeDtypeStruct((B * N, C), jnp.float32),
            jax.ShapeDtypeStruct((B * N, C), jnp.bfloat16),
        ],
    )(flat, g1.reshape(1, C), b1.reshape(1, C), w11, bb11.reshape(1, H),
      w12, bb12.reshape(1, C))

    x1 = x1_flat.reshape(B, N, C)
    x1h = x1h_flat.reshape(B, N, C)

    TR = 512
    rep2 = lambda j, b: (0, 0)
    out = pl.pallas_call(
        _stage2_kernel,
        grid=(N // TR, B),
        in_specs=[
            pl.BlockSpec((TR, N), lambda j, b: (j, 0)),
            pl.BlockSpec((1, N, C), lambda j, b: (b, 0, 0)),
            pl.BlockSpec((1, TR, C), lambda j, b: (b, j, 0)),
            pl.BlockSpec((1, 2 * C), rep2),
            pl.BlockSpec((1, 2 * C), rep2),
            pl.BlockSpec((2 * C, H), rep2),
            pl.BlockSpec((1, H), rep2),
            pl.BlockSpec((H, C), rep2),
            pl.BlockSpec((1, C), rep2),
        ],
        out_specs=pl.BlockSpec((1, TR, C), lambda j, b: (b, j, 0)),
        out_shape=jax.ShapeDtypeStruct((B, N, C), jnp.float32),
    )(adj, x1h, x1, g2.reshape(1, 2 * C), b2.reshape(1, 2 * C), w21,
      bb21.reshape(1, H), w22, bb22.reshape(1, C))

    return (out, edge)


# speedup vs baseline: 3.9038x; 1.1070x over previous
import functools

import jax
import jax.numpy as jnp
from jax.experimental import pallas as pl
from jax.experimental.pallas import tpu as pltpu

_B, _N, _C, _H = 4, 2048, 256, 512
_TN = 1024
_TR = 512
_P1 = (_B * _N) // _TN
_P2 = (_N // _TR) * _B


def _fused_kernel(node_ref, adj_ref, g1_ref, b1_ref, g2_ref, b2_ref,
                  w11_ref, bb11_ref, w12_ref, bb12_ref, w21_ref, bb21_ref,
                  w22_ref, bb22_ref, out_ref, x1_s, x1h_s):
    i = pl.program_id(0)

    @pl.when(i < _P1)
    def _phase1():
        x = node_ref[...]
        m = jnp.mean(x, axis=-1, keepdims=True)
        v = jnp.mean((x - m) ** 2, axis=-1, keepdims=True)
        xn = (x - m) * jax.lax.rsqrt(v + 1e-5) * g1_ref[...] + b1_ref[...]
        h = 0.5 * (t := jnp.dot(xn, w11_ref[...],
                                preferred_element_type=jnp.float32)
                   + bb11_ref[...]) * (1.0 + jax.lax.erf(
                       t * 0.7071067811865476))
        x1 = (jnp.dot(h, w12_ref[...], preferred_element_type=jnp.float32)
              + bb12_ref[...] + x)
        x1_s[pl.ds(i * _TN, _TN), :] = x1
        x1h_s[pl.ds(i * _TN, _TN), :] = x1.astype(jnp.bfloat16)

    @pl.when(i >= _P1)
    def _phase2():
        k = i - _P1
        b = jax.lax.rem(k, _B)
        adj16 = adj_ref[...].astype(jnp.bfloat16)
        agg = jnp.dot(adj16, x1h_s[pl.ds(b * _N, _N), :],
                      preferred_element_type=jnp.float32)
        x1 = x1_s[pl.ds(b * _N + (k // _B) * _TR, _TR), :]
        s = (jnp.sum(x1, axis=-1, keepdims=True)
             + jnp.sum(agg, axis=-1, keepdims=True))
        m = s / (2 * _C)
        q = (jnp.sum((x1 - m) ** 2, axis=-1, keepdims=True)
             + jnp.sum((agg - m) ** 2, axis=-1, keepdims=True))
        rs = jax.lax.rsqrt(q / (2 * _C) + 1e-5)
        g2 = g2_ref[...]
        b2 = b2_ref[...]
        xa = (x1 - m) * rs * g2[:, :_C] + b2[:, :_C]
        xb = (agg - m) * rs * g2[:, _C:] + b2[:, _C:]
        w21 = w21_ref[...]
        t = (jnp.dot(xa, w21[:_C], preferred_element_type=jnp.float32)
             + jnp.dot(xb, w21[_C:], preferred_element_type=jnp.float32)
             + bb21_ref[...])
        h = 0.5 * t * (1.0 + jax.lax.erf(t * 0.7071067811865476))
        out_ref[...] = (jnp.dot(h, w22_ref[...],
                                preferred_element_type=jnp.float32)
                        + bb22_ref[...] + x1)


def _node_map(i):
    return (jnp.minimum(i, _P1 - 1), 0)


def _adj_map(i):
    k = jnp.maximum(i - _P1, 0)
    return (k // _B, 0)


def _out_map(i):
    k = jnp.maximum(i - _P1, 0)
    return (jax.lax.rem(k, _B) * (_N // _TR) + k // _B, 0)


@jax.jit
def kernel(node, edge, adj, g1, b1, g2, b2, w11, bb11, w12, bb12, w21, bb21,
           w22, bb22):
    B, N, C = node.shape
    H = w11.shape[1]
    flat = node.reshape(B * N, C)
    rep = lambda i: (0, 0)

    out_flat = pl.pallas_call(
        _fused_kernel,
        grid=(_P1 + _P2,),
        in_specs=[
            pl.BlockSpec((_TN, C), _node_map),
            pl.BlockSpec((_TR, N), _adj_map),
            pl.BlockSpec((1, C), rep),
            pl.BlockSpec((1, C), rep),
            pl.BlockSpec((1, 2 * C), rep),
            pl.BlockSpec((1, 2 * C), rep),
            pl.BlockSpec((C, H), rep),
            pl.BlockSpec((1, H), rep),
            pl.BlockSpec((H, C), rep),
            pl.BlockSpec((1, C), rep),
            pl.BlockSpec((2 * C, H), rep),
            pl.BlockSpec((1, H), rep),
            pl.BlockSpec((H, C), rep),
            pl.BlockSpec((1, C), rep),
        ],
        out_specs=pl.BlockSpec((_TR, C), _out_map),
        out_shape=jax.ShapeDtypeStruct((B * N, C), jnp.float32),
        scratch_shapes=[
            pltpu.VMEM((B * N, C), jnp.float32),
            pltpu.VMEM((B * N, C), jnp.bfloat16),
        ],
    )(flat, adj, g1.reshape(1, C), b1.reshape(1, C), g2.reshape(1, 2 * C),
      b2.reshape(1, 2 * C), w11, bb11.reshape(1, H), w12, bb12.reshape(1, C),
      w21, bb21.reshape(1, H), w22, bb22.reshape(1, C))

    return (out_flat.reshape(B, N, C), edge)


# speedup vs baseline: 4.2609x; 1.0915x over previous
import functools

import jax
import jax.numpy as jnp
from jax.experimental import pallas as pl
from jax.experimental.pallas import tpu as pltpu

_B, _N, _C, _H = 4, 2048, 256, 512
_TN = 2048
_TR = 1024
_P1 = (_B * _N) // _TN
_P2 = (_N // _TR) * _B


def _fused_kernel(node_ref, adj_ref, g1_ref, b1_ref, g2_ref, b2_ref,
                  w11_ref, bb11_ref, w12_ref, bb12_ref, w21_ref, bb21_ref,
                  w22_ref, bb22_ref, out_ref, x1_s, x1h_s):
    i = pl.program_id(0)

    @pl.when(i < _P1)
    def _phase1():
        x = node_ref[...]
        m = jnp.mean(x, axis=-1, keepdims=True)
        v = jnp.mean((x - m) ** 2, axis=-1, keepdims=True)
        xn = (x - m) * jax.lax.rsqrt(v + 1e-5) * g1_ref[...] + b1_ref[...]
        h = 0.5 * (t := jnp.dot(xn, w11_ref[...],
                                preferred_element_type=jnp.float32)
                   + bb11_ref[...]) * (1.0 + jax.lax.erf(
                       t * 0.7071067811865476))
        x1 = (jnp.dot(h, w12_ref[...], preferred_element_type=jnp.float32)
              + bb12_ref[...] + x)
        x1_s[pl.ds(i * _TN, _TN), :] = x1
        x1h_s[pl.ds(i * _TN, _TN), :] = x1.astype(jnp.bfloat16)

    @pl.when(i >= _P1)
    def _phase2():
        k = i - _P1
        b = jax.lax.rem(k, _B)
        adj16 = adj_ref[...].astype(jnp.bfloat16)
        agg = jnp.dot(adj16, x1h_s[pl.ds(b * _N, _N), :],
                      preferred_element_type=jnp.float32)
        x1 = x1_s[pl.ds(b * _N + (k // _B) * _TR, _TR), :]
        s = (jnp.sum(x1, axis=-1, keepdims=True)
             + jnp.sum(agg, axis=-1, keepdims=True))
        m = s / (2 * _C)
        q = (jnp.sum((x1 - m) ** 2, axis=-1, keepdims=True)
             + jnp.sum((agg - m) ** 2, axis=-1, keepdims=True))
        rs = jax.lax.rsqrt(q / (2 * _C) + 1e-5)
        g2 = g2_ref[...]
        b2 = b2_ref[...]
        xa = (x1 - m) * rs * g2[:, :_C] + b2[:, :_C]
        xb = (agg - m) * rs * g2[:, _C:] + b2[:, _C:]
        w21 = w21_ref[...]
        t = (jnp.dot(xa, w21[:_C], preferred_element_type=jnp.float32)
             + jnp.dot(xb, w21[_C:], preferred_element_type=jnp.float32)
             + bb21_ref[...])
        h = 0.5 * t * (1.0 + jax.lax.erf(t * 0.7071067811865476))
        out_ref[...] = (jnp.dot(h, w22_ref[...],
                                preferred_element_type=jnp.float32)
                        + bb22_ref[...] + x1)


def _node_map(i):
    return (jnp.minimum(i, _P1 - 1), 0)


def _adj_map(i):
    k = jnp.maximum(i - _P1, 0)
    return (k // _B, 0)


def _out_map(i):
    k = jnp.maximum(i - _P1, 0)
    return (jax.lax.rem(k, _B) * (_N // _TR) + k // _B, 0)


@jax.jit
def kernel(node, edge, adj, g1, b1, g2, b2, w11, bb11, w12, bb12, w21, bb21,
           w22, bb22):
    B, N, C = node.shape
    H = w11.shape[1]
    flat = node.reshape(B * N, C)
    rep = lambda i: (0, 0)

    out_flat = pl.pallas_call(
        _fused_kernel,
        grid=(_P1 + _P2,),
        in_specs=[
            pl.BlockSpec((_TN, C), _node_map),
            pl.BlockSpec((_TR, N), _adj_map),
            pl.BlockSpec((1, C), rep),
            pl.BlockSpec((1, C), rep),
            pl.BlockSpec((1, 2 * C), rep),
            pl.BlockSpec((1, 2 * C), rep),
            pl.BlockSpec((C, H), rep),
            pl.BlockSpec((1, H), rep),
            pl.BlockSpec((H, C), rep),
            pl.BlockSpec((1, C), rep),
            pl.BlockSpec((2 * C, H), rep),
            pl.BlockSpec((1, H), rep),
            pl.BlockSpec((H, C), rep),
            pl.BlockSpec((1, C), rep),
        ],
        out_specs=pl.BlockSpec((_TR, C), _out_map),
        out_shape=jax.ShapeDtypeStruct((B * N, C), jnp.float32),
        scratch_shapes=[
            pltpu.VMEM((B * N, C), jnp.float32),
            pltpu.VMEM((B * N, C), jnp.bfloat16),
        ],
    )(flat, adj, g1.reshape(1, C), b1.reshape(1, C), g2.reshape(1, 2 * C),
      b2.reshape(1, 2 * C), w11, bb11.reshape(1, H), w12, bb12.reshape(1, C),
      w21, bb21.reshape(1, H), w22, bb22.reshape(1, C))

    return (out_flat.reshape(B, N, C), edge)
